# baseline (speedup 1.0000x reference)
import functools

import jax
import jax.numpy as jnp
from jax import lax
from jax.experimental import pallas as pl
from jax.experimental.pallas import tpu as pltpu
from jax.experimental.pallas import tpu_sc as plsc

N = 10000
E = 320000
F_IN = 128
HID = 256
OUT = 128

NC, NS = 2, 16
NT = NC * NS
NPAD = 10240
JUNK = NPAD - 1
CH = 128
RPT = E // NT
CPR = 79
RPAD = CPR * CH
BR = 2048
GW = 64
DW = 16
ROWS_PER_TILE = NPAD // NS

_MESH = dict(core_axis_name="c", subcore_axis_name="s",
             num_cores=NC, num_subcores=NS)


def _deg_partials(dst3):
    @functools.partial(
        pl.kernel,
        out_type=jax.ShapeDtypeStruct((NC, NPAD, DW), jnp.float32),
        mesh=plsc.VectorSubcoreMesh(**_MESH),
        scratch_types=[
            pltpu.VMEM((CPR, CH), jnp.int32),
            pltpu.VMEM((CH, DW), jnp.float32),
            pltpu.VMEM_SHARED((NPAD, DW), jnp.float32),
        ],
    )
    def k(dst_hbm, out_hbm, idx_v, buf, deg_sh):
        c = lax.axis_index("c")
        s = lax.axis_index("s")
        r = c * NS + s
        zeros16 = jnp.zeros((16,), jnp.float32)

        def zr(i, _):
            buf[i, pl.ds(0, 16)] = zeros16
            return 0
        lax.fori_loop(0, CH, zr, 0)
        base = s * ROWS_PER_TILE
        for q in range(ROWS_PER_TILE // CH):
            pltpu.sync_copy(buf, deg_sh.at[pl.ds(base + q * CH, CH)])

        ones16 = jnp.ones((16,), jnp.float32)

        def fr(i, _):
            buf[i, pl.ds(0, 16)] = ones16
            return 0
        lax.fori_loop(0, CH, fr, 0)
        pltpu.sync_copy(dst_hbm.at[r], idx_v)
        plsc.subcore_barrier()

        def body(i, _):
            pltpu.sync_copy(buf, deg_sh.at[idx_v.at[i]], add=True)
            return 0
        lax.fori_loop(0, CPR, body, 0)
        plsc.subcore_barrier()
        pltpu.sync_copy(deg_sh.at[pl.ds(base, ROWS_PER_TILE)],
                        out_hbm.at[c].at[pl.ds(base, ROWS_PER_TILE)])

    return k(dst3)


def _edge_pass(src3, dst3, z_split, groups_per_core):
    G = NC * groups_per_core

    @functools.partial(
        pl.kernel,
        out_type=jax.ShapeDtypeStruct((G, NPAD, GW), jnp.float32),
        mesh=plsc.VectorSubcoreMesh(**_MESH),
        compiler_params=pltpu.CompilerParams(use_tc_tiling_on_sc=False),
        scratch_types=(
            [pltpu.VMEM((2 * CPR + 2, CH), jnp.int32),
             pltpu.VMEM((2 * CPR + 2, CH), jnp.int32)]
            + [pltpu.VMEM((CH, GW), jnp.float32)] * 5
            + [pltpu.VMEM_SHARED((NPAD, GW), jnp.float32)]
            + [pltpu.SemaphoreType.DMA] * 4
        ),
    )
    def k(src_hbm, dst_hbm, z_hbm, out_hbm, src_v, dst_v,
          buf0, buf1, buf2, buf3, zbuf, acc_sh, sem0, sem1, sem2, sem3):
        c = lax.axis_index("c")
        s = lax.axis_index("s")
        base = s * ROWS_PER_TILE
        bufs = (buf0, buf1, buf2, buf3)
        sems = (sem0, sem1, sem2, sem3)

        pltpu.sync_copy(src_hbm.at[2 * s], src_v.at[pl.ds(0, CPR)])
        pltpu.sync_copy(src_hbm.at[2 * s + 1], src_v.at[pl.ds(CPR, CPR)])
        pltpu.sync_copy(dst_hbm.at[2 * s], dst_v.at[pl.ds(0, CPR)])
        pltpu.sync_copy(dst_hbm.at[2 * s + 1], dst_v.at[pl.ds(CPR, CPR)])

        zeros16 = jnp.zeros((16,), jnp.float32)
        zeros16i = jnp.zeros((16,), jnp.int32)
        junk16 = jnp.full((16,), JUNK, jnp.int32)

        def zr(i, _):
            def zc(l, _):
                zbuf[i, pl.ds(l * 16, 16)] = zeros16
                return 0
            return lax.fori_loop(0, GW // 16, zc, 0)
        lax.fori_loop(0, CH, zr, 0)
        for l in range(CH // 16):
            src_v[2 * CPR, pl.ds(l * 16, 16)] = zeros16i
            src_v[2 * CPR + 1, pl.ds(l * 16, 16)] = zeros16i
            dst_v[2 * CPR, pl.ds(l * 16, 16)] = junk16
            dst_v[2 * CPR + 1, pl.ds(l * 16, 16)] = junk16

        END = 2 * CPR + 2

        def do_group(grp):
            for q in range(ROWS_PER_TILE // CH):
                pltpu.sync_copy(zbuf, acc_sh.at[pl.ds(base + q * CH, CH)])
            plsc.subcore_barrier()

            zg = z_hbm.at[grp]
            for b in range(4):
                pltpu.async_copy(zg.at[src_v.at[b]], bufs[b], sems[b])

            def body(i, _):
                for b in range(4):
                    kk = 4 * i + b
                    pltpu.make_async_copy(zg.at[src_v.at[0]], bufs[b],
                                          sems[b]).wait()
                    pltpu.sync_copy(bufs[b], acc_sh.at[dst_v.at[kk]],
                                    add=True)

                    @pl.when(kk + 4 < END)
                    def _():
                        pltpu.async_copy(zg.at[src_v.at[kk + 4]],
                                         bufs[b], sems[b])
                return 0
            lax.fori_loop(0, END // 4, body, 0)
            plsc.subcore_barrier()
            pltpu.sync_copy(acc_sh.at[pl.ds(base, ROWS_PER_TILE)],
                            out_hbm.at[grp].at[pl.ds(base, ROWS_PER_TILE)])

        for cc in range(NC):
            @pl.when(c == cc)
            def _():
                for g in range(groups_per_core):
                    do_group(cc * groups_per_core + g)

    return k(src3, dst3, z_split)


def _scaled_matmul1(xp, W1g, deg_parts):
    def body(x_ref, w_ref, degp_ref, out_ref):
        p = degp_ref[...]
        d = lax.rsqrt(1.0 + p[0, :, 0] + p[1, :, 0])
        xw = jnp.dot(x_ref[...], w_ref[0],
                     preferred_element_type=jnp.float32)
        out_ref[...] = (xw * d[:, None])[None]

    return pl.pallas_call(
        body,
        grid=(NPAD // BR, HID // GW),
        in_specs=[
            pl.BlockSpec((BR, F_IN), lambda i, j: (i, 0)),
            pl.BlockSpec((1, F_IN, GW), lambda i, j: (j, 0, 0)),
            pl.BlockSpec((NC, BR, DW), lambda i, j: (0, i, 0)),
        ],
        out_specs=pl.BlockSpec((1, BR, GW), lambda i, j: (j, i, 0)),
        out_shape=jax.ShapeDtypeStruct((HID // GW, NPAD, GW), jnp.float32),
    )(xp, W1g, deg_parts)


def _hidden_matmul2(acc1f, z1f, deg_parts, b1r, W2):
    def body(a_ref, z_ref, degp_ref, b1_ref, w2_ref, out_ref):
        p = degp_ref[...]
        d = lax.rsqrt(1.0 + p[0, :, 0] + p[1, :, 0])[:, None]
        h = (a_ref[...] + z_ref[...]) * d + b1_ref[...]
        h = jnp.maximum(h, 0.0)
        y = jnp.dot(h, w2_ref[...], preferred_element_type=jnp.float32)
        out_ref[...] = y * d

    return pl.pallas_call(
        body,
        grid=(NPAD // BR,),
        in_specs=[
            pl.BlockSpec((BR, HID), lambda i: (i, 0)),
            pl.BlockSpec((BR, HID), lambda i: (i, 0)),
            pl.BlockSpec((NC, BR, DW), lambda i: (0, i, 0)),
            pl.BlockSpec((1, HID), lambda i: (0, 0)),
            pl.BlockSpec((HID, OUT), lambda i: (0, 0)),
        ],
        out_specs=pl.BlockSpec((BR, OUT), lambda i: (i, 0)),
        out_shape=jax.ShapeDtypeStruct((NPAD, OUT), jnp.float32),
    )(acc1f, z1f, deg_parts, b1r, W2)


def _finalize(acc2f, z2f, deg_parts, b2r):
    def body(a_ref, z_ref, degp_ref, b2_ref, out_ref):
        p = degp_ref[...]
        d = lax.rsqrt(1.0 + p[0, :, 0] + p[1, :, 0])[:, None]
        out_ref[...] = (a_ref[...] + z_ref[...]) * d + b2_ref[...]

    return pl.pallas_call(
        body,
        grid=(NPAD // BR,),
        in_specs=[
            pl.BlockSpec((BR, OUT), lambda i: (i, 0)),
            pl.BlockSpec((BR, OUT), lambda i: (i, 0)),
            pl.BlockSpec((NC, BR, DW), lambda i: (0, i, 0)),
            pl.BlockSpec((1, OUT), lambda i: (0, 0)),
        ],
        out_specs=pl.BlockSpec((BR, OUT), lambda i: (i, 0)),
        out_shape=jax.ShapeDtypeStruct((N, OUT), jnp.float32),
    )(acc2f, z2f, deg_parts, b2r)


def kernel(x, edge_index, W1, b1, W2, b2):
    src = edge_index[0].reshape(NT, RPT)
    dst = edge_index[1].reshape(NT, RPT)
    pad = RPAD - RPT
    src3 = jnp.pad(src, ((0, 0), (0, pad)),
                   constant_values=0).reshape(NT, CPR, CH)
    dst3 = jnp.pad(dst, ((0, 0), (0, pad)),
                   constant_values=JUNK).reshape(NT, CPR, CH)
    xp = jnp.pad(x, ((0, NPAD - N), (0, 0)))

    W1g = jnp.transpose(W1.reshape(F_IN, HID // GW, GW), (1, 0, 2))

    deg_parts = _deg_partials(dst3)
    z1 = _scaled_matmul1(xp, W1g, deg_parts)
    acc1 = _edge_pass(src3, dst3, z1, 2)
    z1f = jnp.transpose(z1, (1, 0, 2)).reshape(NPAD, HID)
    acc1f = jnp.transpose(acc1, (1, 0, 2)).reshape(NPAD, HID)
    z2f = _hidden_matmul2(acc1f, z1f, deg_parts,
                          b1.reshape(1, HID), W2)
    z2 = jnp.transpose(z2f.reshape(NPAD, OUT // GW, GW), (1, 0, 2))
    acc2 = _edge_pass(src3, dst3, z2, 1)
    acc2f = jnp.transpose(acc2, (1, 0, 2)).reshape(NPAD, OUT)
    return _finalize(acc2f, z2f, deg_parts, b2.reshape(1, OUT))


# speedup vs baseline: 1.1989x; 1.1989x over previous
import functools

import jax
import jax.numpy as jnp
from jax import lax
from jax.experimental import pallas as pl
from jax.experimental.pallas import tpu as pltpu
from jax.experimental.pallas import tpu_sc as plsc

N = 10000
E = 320000
F_IN = 128
HID = 256
OUT = 128

NC, NS = 2, 16
NT = NC * NS
NPAD = 10240
JUNK = NPAD - 1
CH = 128
RPT = E // NT
CPR = 79
RPAD = CPR * CH
BR = 2048
GW = 64
DW = 16
ROWS_PER_TILE = NPAD // NS

_MESH = dict(core_axis_name="c", subcore_axis_name="s",
             num_cores=NC, num_subcores=NS)


def _deg_partials(dst3):
    @functools.partial(
        pl.kernel,
        out_type=jax.ShapeDtypeStruct((NC, NPAD, DW), jnp.float32),
        mesh=plsc.VectorSubcoreMesh(**_MESH),
        scratch_types=[
            pltpu.VMEM((CPR, CH), jnp.int32),
            pltpu.VMEM((CH, DW), jnp.float32),
            pltpu.VMEM_SHARED((NPAD, DW), jnp.float32),
        ],
    )
    def k(dst_hbm, out_hbm, idx_v, buf, deg_sh):
        c = lax.axis_index("c")
        s = lax.axis_index("s")
        r = c * NS + s
        zeros16 = jnp.zeros((16,), jnp.float32)

        def zr(i, _):
            buf[i, pl.ds(0, 16)] = zeros16
            return 0
        lax.fori_loop(0, CH, zr, 0)
        base = s * ROWS_PER_TILE
        for q in range(ROWS_PER_TILE // CH):
            pltpu.sync_copy(buf, deg_sh.at[pl.ds(base + q * CH, CH)])

        ones16 = jnp.ones((16,), jnp.float32)

        def fr(i, _):
            buf[i, pl.ds(0, 16)] = ones16
            return 0
        lax.fori_loop(0, CH, fr, 0)
        pltpu.sync_copy(dst_hbm.at[r], idx_v)
        plsc.subcore_barrier()

        def body(i, _):
            pltpu.sync_copy(buf, deg_sh.at[idx_v.at[i]], add=True)
            return 0
        lax.fori_loop(0, CPR, body, 0)
        plsc.subcore_barrier()
        pltpu.sync_copy(deg_sh.at[pl.ds(base, ROWS_PER_TILE)],
                        out_hbm.at[c].at[pl.ds(base, ROWS_PER_TILE)])

    return k(dst3)


def _edge_pass(src3, dst3, z_split, groups_per_core):
    G = NC * groups_per_core

    @functools.partial(
        pl.kernel,
        out_type=jax.ShapeDtypeStruct((G, NPAD, GW), jnp.float32),
        mesh=plsc.VectorSubcoreMesh(**_MESH),
        compiler_params=pltpu.CompilerParams(use_tc_tiling_on_sc=False),
        scratch_types=[
            pltpu.VMEM((2 * CPR, CH), jnp.int32),
            pltpu.VMEM((2 * CPR, CH), jnp.int32),
            pltpu.VMEM((CH, GW), jnp.float32),
            pltpu.VMEM((CH, GW), jnp.float32),
            pltpu.VMEM((CH, GW), jnp.float32),
            pltpu.VMEM_SHARED((NPAD, GW), jnp.float32),
            pltpu.SemaphoreType.DMA,
            pltpu.SemaphoreType.DMA,
        ],
    )
    def k(src_hbm, dst_hbm, z_hbm, out_hbm, src_v, dst_v,
          buf0, buf1, zbuf, acc_sh, sem0, sem1):
        c = lax.axis_index("c")
        s = lax.axis_index("s")
        base = s * ROWS_PER_TILE

        pltpu.sync_copy(src_hbm.at[2 * s], src_v.at[pl.ds(0, CPR)])
        pltpu.sync_copy(src_hbm.at[2 * s + 1], src_v.at[pl.ds(CPR, CPR)])
        pltpu.sync_copy(dst_hbm.at[2 * s], dst_v.at[pl.ds(0, CPR)])
        pltpu.sync_copy(dst_hbm.at[2 * s + 1], dst_v.at[pl.ds(CPR, CPR)])

        zeros16 = jnp.zeros((16,), jnp.float32)

        def zr(i, _):
            def zc(l, _):
                zbuf[i, pl.ds(l * 16, 16)] = zeros16
                return 0
            return lax.fori_loop(0, GW // 16, zc, 0)
        lax.fori_loop(0, CH, zr, 0)

        END = 2 * CPR

        def do_group(grp):
            for q in range(ROWS_PER_TILE // CH):
                pltpu.sync_copy(zbuf, acc_sh.at[pl.ds(base + q * CH, CH)])
            plsc.subcore_barrier()

            zg = z_hbm.at[grp]
            pltpu.async_copy(zg.at[src_v.at[0]], buf0, sem0)
            pltpu.async_copy(zg.at[src_v.at[1]], buf1, sem1)

            def body(i, _):
                kk = 2 * i
                pltpu.make_async_copy(zg.at[src_v.at[0]], buf0, sem0).wait()
                pltpu.sync_copy(buf0, acc_sh.at[dst_v.at[kk]], add=True)

                @pl.when(kk + 2 < END)
                def _():
                    pltpu.async_copy(zg.at[src_v.at[kk + 2]], buf0, sem0)
                pltpu.make_async_copy(zg.at[src_v.at[1]], buf1, sem1).wait()
                pltpu.sync_copy(buf1, acc_sh.at[dst_v.at[kk + 1]], add=True)

                @pl.when(kk + 3 < END)
                def _():
                    pltpu.async_copy(zg.at[src_v.at[kk + 3]], buf1, sem1)
                return 0
            lax.fori_loop(0, END // 2, body, 0)
            plsc.subcore_barrier()
            pltpu.sync_copy(acc_sh.at[pl.ds(base, ROWS_PER_TILE)],
                            out_hbm.at[grp].at[pl.ds(base, ROWS_PER_TILE)])

        for cc in range(NC):
            @pl.when(c == cc)
            def _():
                for g in range(groups_per_core):
                    do_group(cc * groups_per_core + g)

    return k(src3, dst3, z_split)


def _scaled_matmul1(xp, W1g, deg_parts):
    def body(x_ref, w_ref, degp_ref, out_ref):
        p = degp_ref[...]
        d = lax.rsqrt(1.0 + p[0, :, 0] + p[1, :, 0])
        xw = jnp.dot(x_ref[...], w_ref[0],
                     preferred_element_type=jnp.float32)
        out_ref[...] = (xw * d[:, None])[None]

    return pl.pallas_call(
        body,
        grid=(NPAD // BR, HID // GW),
        in_specs=[
            pl.BlockSpec((BR, F_IN), lambda i, j: (i, 0)),
            pl.BlockSpec((1, F_IN, GW), lambda i, j: (j, 0, 0)),
            pl.BlockSpec((NC, BR, DW), lambda i, j: (0, i, 0)),
        ],
        out_specs=pl.BlockSpec((1, BR, GW), lambda i, j: (j, i, 0)),
        out_shape=jax.ShapeDtypeStruct((HID // GW, NPAD, GW), jnp.float32),
    )(xp, W1g, deg_parts)


def _hidden_matmul2(acc1f, z1f, deg_parts, b1r, W2):
    def body(a_ref, z_ref, degp_ref, b1_ref, w2_ref, out_ref):
        p = degp_ref[...]
        d = lax.rsqrt(1.0 + p[0, :, 0] + p[1, :, 0])[:, None]
        h = (a_ref[...] + z_ref[...]) * d + b1_ref[...]
        h = jnp.maximum(h, 0.0)
        y = jnp.dot(h, w2_ref[...], preferred_element_type=jnp.float32)
        out_ref[...] = y * d

    return pl.pallas_call(
        body,
        grid=(NPAD // BR,),
        in_specs=[
            pl.BlockSpec((BR, HID), lambda i: (i, 0)),
            pl.BlockSpec((BR, HID), lambda i: (i, 0)),
            pl.BlockSpec((NC, BR, DW), lambda i: (0, i, 0)),
            pl.BlockSpec((1, HID), lambda i: (0, 0)),
            pl.BlockSpec((HID, OUT), lambda i: (0, 0)),
        ],
        out_specs=pl.BlockSpec((BR, OUT), lambda i: (i, 0)),
        out_shape=jax.ShapeDtypeStruct((NPAD, OUT), jnp.float32),
    )(acc1f, z1f, deg_parts, b1r, W2)


def _finalize(acc2f, z2f, deg_parts, b2r):
    def body(a_ref, z_ref, degp_ref, b2_ref, out_ref):
        p = degp_ref[...]
        d = lax.rsqrt(1.0 + p[0, :, 0] + p[1, :, 0])[:, None]
        out_ref[...] = (a_ref[...] + z_ref[...]) * d + b2_ref[...]

    return pl.pallas_call(
        body,
        grid=(NPAD // BR,),
        in_specs=[
            pl.BlockSpec((BR, OUT), lambda i: (i, 0)),
            pl.BlockSpec((BR, OUT), lambda i: (i, 0)),
            pl.BlockSpec((NC, BR, DW), lambda i: (0, i, 0)),
            pl.BlockSpec((1, OUT), lambda i: (0, 0)),
        ],
        out_specs=pl.BlockSpec((BR, OUT), lambda i: (i, 0)),
        out_shape=jax.ShapeDtypeStruct((N, OUT), jnp.float32),
    )(acc2f, z2f, deg_parts, b2r)


def kernel(x, edge_index, W1, b1, W2, b2):
    src = edge_index[0].reshape(NT, RPT)
    dst = edge_index[1].reshape(NT, RPT)
    pad = RPAD - RPT
    src3 = jnp.pad(src, ((0, 0), (0, pad)),
                   constant_values=0).reshape(NT, CPR, CH)
    dst3 = jnp.pad(dst, ((0, 0), (0, pad)),
                   constant_values=JUNK).reshape(NT, CPR, CH)
    xp = jnp.pad(x, ((0, NPAD - N), (0, 0)))

    W1g = jnp.transpose(W1.reshape(F_IN, HID // GW, GW), (1, 0, 2))

    deg_parts = _deg_partials(dst3)
    z1 = _scaled_matmul1(xp, W1g, deg_parts)
    acc1 = _edge_pass(src3, dst3, z1, 2)
    z1f = jnp.transpose(z1, (1, 0, 2)).reshape(NPAD, HID)
    acc1f = jnp.transpose(acc1, (1, 0, 2)).reshape(NPAD, HID)
    z2f = _hidden_matmul2(acc1f, z1f, deg_parts,
                          b1.reshape(1, HID), W2)
    z2 = jnp.transpose(z2f.reshape(NPAD, OUT // GW, GW), (1, 0, 2))
    acc2 = _edge_pass(src3, dst3, z2, 1)
    acc2f = jnp.transpose(acc2, (1, 0, 2)).reshape(NPAD, OUT)
    return _finalize(acc2f, z2f, deg_parts, b2.reshape(1, OUT))


# speedup vs baseline: 1.3012x; 1.0853x over previous
import functools

import jax
import jax.numpy as jnp
from jax import lax
from jax.experimental import pallas as pl
from jax.experimental.pallas import tpu as pltpu
from jax.experimental.pallas import tpu_sc as plsc

N = 10000
E = 320000
F_IN = 128
HID = 256
OUT = 128

NC, NS = 2, 16
NT = NC * NS
NPAD = 10240
JUNK = NPAD - 1
CH = 128
RPT = E // NT
CPR = 79
RPAD = CPR * CH
BR = 2048
GW = 64
DW = 16
ROWS_PER_TILE = NPAD // NS

_MESH = dict(core_axis_name="c", subcore_axis_name="s",
             num_cores=NC, num_subcores=NS)


def _deg_partials(dst3):
    @functools.partial(
        pl.kernel,
        out_type=jax.ShapeDtypeStruct((NC, NPAD, DW), jnp.float32),
        mesh=plsc.VectorSubcoreMesh(**_MESH),
        scratch_types=[
            pltpu.VMEM((CPR, CH), jnp.int32),
            pltpu.VMEM((CH, DW), jnp.float32),
            pltpu.VMEM_SHARED((NPAD, DW), jnp.float32),
        ],
    )
    def k(dst_hbm, out_hbm, idx_v, buf, deg_sh):
        c = lax.axis_index("c")
        s = lax.axis_index("s")
        r = c * NS + s
        zeros16 = jnp.zeros((16,), jnp.float32)

        def zr(i, _):
            buf[i, pl.ds(0, 16)] = zeros16
            return 0
        lax.fori_loop(0, CH, zr, 0)
        base = s * ROWS_PER_TILE
        for q in range(ROWS_PER_TILE // CH):
            pltpu.sync_copy(buf, deg_sh.at[pl.ds(base + q * CH, CH)])

        ones16 = jnp.ones((16,), jnp.float32)

        def fr(i, _):
            buf[i, pl.ds(0, 16)] = ones16
            return 0
        lax.fori_loop(0, CH, fr, 0)
        pltpu.sync_copy(dst_hbm.at[r], idx_v)
        plsc.subcore_barrier()

        def body(i, _):
            pltpu.sync_copy(buf, deg_sh.at[idx_v.at[i]], add=True)
            return 0
        lax.fori_loop(0, CPR, body, 0)
        plsc.subcore_barrier()
        pltpu.sync_copy(deg_sh.at[pl.ds(base, ROWS_PER_TILE)],
                        out_hbm.at[c].at[pl.ds(base, ROWS_PER_TILE)])

    return k(dst3)


def _edge_pass(src3, dst3, z_split, groups_per_core):
    G = NC * groups_per_core

    @functools.partial(
        pl.kernel,
        out_type=jax.ShapeDtypeStruct((NPAD, G * GW), jnp.float32),
        mesh=plsc.VectorSubcoreMesh(**_MESH),
        compiler_params=pltpu.CompilerParams(use_tc_tiling_on_sc=False),
        scratch_types=[
            pltpu.VMEM((2 * CPR, CH), jnp.int32),
            pltpu.VMEM((2 * CPR, CH), jnp.int32),
            pltpu.VMEM((CH, GW), jnp.float32),
            pltpu.VMEM((CH, GW), jnp.float32),
            pltpu.VMEM((CH, GW), jnp.float32),
            pltpu.VMEM_SHARED((NPAD, GW), jnp.float32),
            pltpu.SemaphoreType.DMA,
            pltpu.SemaphoreType.DMA,
        ],
    )
    def k(src_hbm, dst_hbm, z_hbm, out_hbm, src_v, dst_v,
          buf0, buf1, zbuf, acc_sh, sem0, sem1):
        c = lax.axis_index("c")
        s = lax.axis_index("s")
        base = s * ROWS_PER_TILE

        pltpu.sync_copy(src_hbm.at[2 * s], src_v.at[pl.ds(0, CPR)])
        pltpu.sync_copy(src_hbm.at[2 * s + 1], src_v.at[pl.ds(CPR, CPR)])
        pltpu.sync_copy(dst_hbm.at[2 * s], dst_v.at[pl.ds(0, CPR)])
        pltpu.sync_copy(dst_hbm.at[2 * s + 1], dst_v.at[pl.ds(CPR, CPR)])

        zeros16 = jnp.zeros((16,), jnp.float32)

        def zr(i, _):
            def zc(l, _):
                zbuf[i, pl.ds(l * 16, 16)] = zeros16
                return 0
            return lax.fori_loop(0, GW // 16, zc, 0)
        lax.fori_loop(0, CH, zr, 0)

        END = 2 * CPR

        def do_group(grp):
            for q in range(ROWS_PER_TILE // CH):
                pltpu.sync_copy(zbuf, acc_sh.at[pl.ds(base + q * CH, CH)])
            plsc.subcore_barrier()

            zg = z_hbm.at[grp]
            pltpu.async_copy(zg.at[src_v.at[0]], buf0, sem0)
            pltpu.async_copy(zg.at[src_v.at[1]], buf1, sem1)

            def body(i, _):
                kk = 2 * i
                pltpu.make_async_copy(zg.at[src_v.at[0]], buf0, sem0).wait()
                pltpu.sync_copy(buf0, acc_sh.at[dst_v.at[kk]], add=True)

                @pl.when(kk + 2 < END)
                def _():
                    pltpu.async_copy(zg.at[src_v.at[kk + 2]], buf0, sem0)
                pltpu.make_async_copy(zg.at[src_v.at[1]], buf1, sem1).wait()
                pltpu.sync_copy(buf1, acc_sh.at[dst_v.at[kk + 1]], add=True)

                @pl.when(kk + 3 < END)
                def _():
                    pltpu.async_copy(zg.at[src_v.at[kk + 3]], buf1, sem1)
                return 0
            lax.fori_loop(0, END // 2, body, 0)
            plsc.subcore_barrier()
            pltpu.sync_copy(acc_sh.at[pl.ds(base, ROWS_PER_TILE)],
                            out_hbm.at[pl.ds(base, ROWS_PER_TILE),
                                       pl.ds(grp * GW, GW)])

        for cc in range(NC):
            @pl.when(c == cc)
            def _():
                for g in range(groups_per_core):
                    do_group(cc * groups_per_core + g)

    return k(src3, dst3, z_split)


def _scaled_matmul1(xp, W1g, deg_parts):
    def body(x_ref, w_ref, degp_ref, out_ref, outf_ref):
        p = degp_ref[...]
        d = lax.rsqrt(1.0 + p[0, :, 0] + p[1, :, 0])
        xw = jnp.dot(x_ref[...], w_ref[0],
                     preferred_element_type=jnp.float32)
        z = xw * d[:, None]
        out_ref[...] = jnp.stack([z[:, :GW], z[:, GW:]], axis=0)
        outf_ref[...] = z

    return pl.pallas_call(
        body,
        grid=(NPAD // BR, HID // (2 * GW)),
        in_specs=[
            pl.BlockSpec((BR, F_IN), lambda i, j: (i, 0)),
            pl.BlockSpec((1, F_IN, 2 * GW), lambda i, j: (j, 0, 0)),
            pl.BlockSpec((NC, BR, DW), lambda i, j: (0, i, 0)),
        ],
        out_specs=[
            pl.BlockSpec((2, BR, GW), lambda i, j: (j, i, 0)),
            pl.BlockSpec((BR, 2 * GW), lambda i, j: (i, j)),
        ],
        out_shape=[
            jax.ShapeDtypeStruct((HID // GW, NPAD, GW), jnp.float32),
            jax.ShapeDtypeStruct((NPAD, HID), jnp.float32),
        ],
    )(xp, W1g, deg_parts)


def _hidden_matmul2(acc1f, z1f, deg_parts, b1r, W2):
    def body(a_ref, z_ref, degp_ref, b1_ref, w2_ref, out_ref, outs_ref):
        p = degp_ref[...]
        d = lax.rsqrt(1.0 + p[0, :, 0] + p[1, :, 0])[:, None]
        h = (a_ref[...] + z_ref[...]) * d + b1_ref[...]
        h = jnp.maximum(h, 0.0)
        y = jnp.dot(h, w2_ref[...], preferred_element_type=jnp.float32)
        y = y * d
        out_ref[...] = y
        outs_ref[...] = jnp.stack([y[:, :GW], y[:, GW:]], axis=0)

    return pl.pallas_call(
        body,
        grid=(NPAD // BR,),
        in_specs=[
            pl.BlockSpec((BR, HID), lambda i: (i, 0)),
            pl.BlockSpec((BR, HID), lambda i: (i, 0)),
            pl.BlockSpec((NC, BR, DW), lambda i: (0, i, 0)),
            pl.BlockSpec((1, HID), lambda i: (0, 0)),
            pl.BlockSpec((HID, OUT), lambda i: (0, 0)),
        ],
        out_specs=[
            pl.BlockSpec((BR, OUT), lambda i: (i, 0)),
            pl.BlockSpec((OUT // GW, BR, GW), lambda i: (0, i, 0)),
        ],
        out_shape=[
            jax.ShapeDtypeStruct((NPAD, OUT), jnp.float32),
            jax.ShapeDtypeStruct((OUT // GW, NPAD, GW), jnp.float32),
        ],
    )(acc1f, z1f, deg_parts, b1r, W2)


def _finalize(acc2f, z2f, deg_parts, b2r):
    def body(a_ref, z_ref, degp_ref, b2_ref, out_ref):
        p = degp_ref[...]
        d = lax.rsqrt(1.0 + p[0, :, 0] + p[1, :, 0])[:, None]
        out_ref[...] = (a_ref[...] + z_ref[...]) * d + b2_ref[...]

    return pl.pallas_call(
        body,
        grid=(NPAD // BR,),
        in_specs=[
            pl.BlockSpec((BR, OUT), lambda i: (i, 0)),
            pl.BlockSpec((BR, OUT), lambda i: (i, 0)),
            pl.BlockSpec((NC, BR, DW), lambda i: (0, i, 0)),
            pl.BlockSpec((1, OUT), lambda i: (0, 0)),
        ],
        out_specs=pl.BlockSpec((BR, OUT), lambda i: (i, 0)),
        out_shape=jax.ShapeDtypeStruct((N, OUT), jnp.float32),
    )(acc2f, z2f, deg_parts, b2r)


def kernel(x, edge_index, W1, b1, W2, b2):
    src = edge_index[0].reshape(NT, RPT)
    dst = edge_index[1].reshape(NT, RPT)
    pad = RPAD - RPT
    src3 = jnp.pad(src, ((0, 0), (0, pad)),
                   constant_values=0).reshape(NT, CPR, CH)
    dst3 = jnp.pad(dst, ((0, 0), (0, pad)),
                   constant_values=JUNK).reshape(NT, CPR, CH)
    xp = jnp.pad(x, ((0, NPAD - N), (0, 0)))

    W1g = jnp.transpose(W1.reshape(F_IN, HID // (2 * GW), 2 * GW),
                        (1, 0, 2))

    deg_parts = _deg_partials(dst3)
    z1, z1f = _scaled_matmul1(xp, W1g, deg_parts)
    acc1f = _edge_pass(src3, dst3, z1, 2)
    z2f, z2 = _hidden_matmul2(acc1f, z1f, deg_parts,
                              b1.reshape(1, HID), W2)
    acc2f = _edge_pass(src3, dst3, z2, 1)
    return _finalize(acc2f, z2f, deg_parts, b2.reshape(1, OUT))
